# baseline (speedup 1.0000x reference)
import functools

import jax
import jax.numpy as jnp
from jax import lax
from jax.experimental import pallas as pl
from jax.experimental.pallas import tpu as pltpu
from jax.experimental.pallas import tpu_sc as plsc

NC = 2
NS = 16
LANES = 16
NW = NC * NS

B = 1024
SEQ = 200
D = 128
DCH = D // LANES

ROWS_PER_W = B // NW
FLAT_PER_W = ROWS_PER_W * SEQ
VCH_PER_ROW = -(-SEQ // LANES)
IDX_PAD = VCH_PER_ROW * LANES - SEQ
ZROWS = 128
BSEG = 8
NBB = 4
BITS = (128, 64, 32, 16, 8)
DRAIN_UNITS = ROWS_PER_W * (SEQ - BSEG) // ZROWS


def _vgather16(vec, idx):
    dnums = lax.GatherDimensionNumbers(
        offset_dims=(), collapsed_slice_dims=(0,), start_index_map=(0,)
    )
    return lax.gather(
        vec, idx[:, None], dnums, slice_sizes=(1,),
        mode=lax.GatherScatterMode.PROMISE_IN_BOUNDS,
    )


def _m8(x):
    return pl.multiple_of(x, 8)


def _sc_body(len_hbm, pe_hbm, out_hbm, pos_hbm, len_v, tz_v, idx_v,
             bbufs, dummy_v, psem, dsem, bsems):
    wid = lax.axis_index("s") * NC + lax.axis_index("c")
    flat_base = _m8(wid * FLAT_PER_W)
    row_base = _m8(wid * ROWS_PER_W)

    pltpu.sync_copy(len_hbm.at[pl.ds(row_base, ROWS_PER_W)],
                    len_v.at[pl.ds(0, ROWS_PER_W)])
    tstage = pltpu.async_copy(pe_hbm, tz_v.at[pl.ds(0, SEQ)], psem)

    iota = lax.iota(jnp.int32, LANES)
    zero16 = jnp.zeros((LANES,), jnp.float32)

    def zrow(r, carry):
        for c in range(DCH):
            tz_v[SEQ + r, pl.ds(c * LANES, LANES)] = zero16
        return carry

    lax.fori_loop(0, ZROWS, zrow, 0)

    lo16 = len_v[pl.ds(0, LANES)]
    hi16 = len_v[pl.ds(LANES, LANES)]

    def fill_row(r, carry):
        r_lane = jnp.full((LANES,), r, jnp.int32)
        lens = jnp.where(
            r < LANES,
            _vgather16(lo16, jnp.minimum(r_lane, LANES - 1)),
            _vgather16(hi16, jnp.maximum(r_lane - LANES, 0)),
        )
        for jc in range(VCH_PER_ROW):
            j1 = jc * LANES + iota + 1
            idx_v[pl.ds(r * SEQ + jc * LANES, LANES)] = jnp.where(
                j1 <= lens, j1, 0
            )
        return carry

    lax.fori_loop(0, ROWS_PER_W, fill_row, 0)

    tstage.wait()

    pltpu.async_copy(idx_v.at[pl.ds(0, FLAT_PER_W)],
                     pos_hbm.at[pl.ds(flat_base, FLAT_PER_W)], psem)

    def row_body(r, carry):
        len_s = len_v[pl.ds(r, LANES)][0]
        q8 = _m8(lax.bitwise_and(len_s, -BSEG))
        s = lax.bitwise_and(len_s, BSEG - 1)
        rowflat = _m8(flat_base + r * SEQ)

        for sz in BITS:
            off = _m8(lax.bitwise_and(q8, -(2 * sz)))

            @pl.when(lax.bitwise_and(q8, sz) != 0)
            def _():
                pltpu.async_copy(
                    tz_v.at[pl.ds(off, sz)],
                    out_hbm.at[pl.ds(_m8(rowflat + off), sz)],
                    dsem,
                )

        for bb in range(NBB):
            @pl.when(lax.rem(r, NBB) == bb)
            def _():
                @pl.when(r >= NBB)
                def _():
                    pltpu.make_async_copy(
                        bbufs[bb],
                        out_hbm.at[pl.ds(flat_base, BSEG)],
                        bsems[bb],
                    ).wait()

                for i in range(BSEG):
                    for c in range(DCH):
                        tvec = tz_v[q8 + i, pl.ds(c * LANES, LANES)]
                        bbufs[bb][i, pl.ds(c * LANES, LANES)] = jnp.where(
                            i < s, tvec, zero16
                        )
                pltpu.async_copy(
                    bbufs[bb],
                    out_hbm.at[pl.ds(_m8(rowflat + q8), BSEG)],
                    bsems[bb],
                )

        z = (SEQ - BSEG) - q8
        for sz in BITS:
            zoff = q8 + BSEG + lax.bitwise_and(z, -(2 * sz))

            @pl.when(lax.bitwise_and(z, sz) != 0)
            def _():
                pltpu.async_copy(
                    tz_v.at[pl.ds(SEQ, sz)],
                    out_hbm.at[pl.ds(_m8(rowflat + zoff), sz)],
                    dsem,
                )

        return carry

    lax.fori_loop(0, ROWS_PER_W, row_body, 0)

    def drain(i, carry):
        pltpu.make_async_copy(
            out_hbm.at[pl.ds(flat_base, ZROWS)], dummy_v, dsem
        ).wait()
        return carry

    lax.fori_loop(0, DRAIN_UNITS, drain, 0)

    for bb in range(NBB):
        pltpu.make_async_copy(
            bbufs[bb], out_hbm.at[pl.ds(flat_base, BSEG)], bsems[bb]
        ).wait()

    pltpu.make_async_copy(
        idx_v.at[pl.ds(0, FLAT_PER_W)],
        pos_hbm.at[pl.ds(flat_base, FLAT_PER_W)], psem
    ).wait()


@functools.partial(jax.jit, static_argnames=())
def _run(lens, pe_shift):
    mesh = plsc.VectorSubcoreMesh(
        core_axis_name="c", subcore_axis_name="s", num_cores=NC, num_subcores=NS
    )
    out_flat, pos_flat = pl.kernel(
        _sc_body,
        out_type=[
            jax.ShapeDtypeStruct((B * SEQ, D), jnp.float32),
            jax.ShapeDtypeStruct((B * SEQ,), jnp.int32),
        ],
        mesh=mesh,
        scratch_types=[
            pltpu.VMEM((ROWS_PER_W + LANES,), jnp.int32),
            pltpu.VMEM((SEQ + ZROWS, D), jnp.float32),
            pltpu.VMEM((FLAT_PER_W + IDX_PAD,), jnp.int32),
            [pltpu.VMEM((BSEG, D), jnp.float32) for _ in range(NBB)],
            pltpu.VMEM((ZROWS, D), jnp.float32),
            pltpu.SemaphoreType.DMA,
            pltpu.SemaphoreType.DMA,
            [pltpu.SemaphoreType.DMA for _ in range(NBB)],
        ],
    )(lens, pe_shift)
    return out_flat, pos_flat


def kernel(input_len, position_encoding):
    lens = input_len.astype(jnp.int32)
    pe_shift = position_encoding[1:]
    out_flat, pos_flat = _run(lens, pe_shift)
    positions = out_flat.reshape(B, SEQ, D)
    input_pos = pos_flat.reshape(B, SEQ)
    return positions, input_pos


# speedup vs baseline: 1.0199x; 1.0199x over previous
import functools

import jax
import jax.numpy as jnp
from jax import lax
from jax.experimental import pallas as pl
from jax.experimental.pallas import tpu as pltpu
from jax.experimental.pallas import tpu_sc as plsc

NC = 2
NS = 16
LANES = 16
NW = NC * NS

B = 1024
SEQ = 200
D = 128
DCH = D // LANES

ROWS_PER_W = B // NW
FLAT_PER_W = ROWS_PER_W * SEQ
VCH_PER_ROW = -(-SEQ // LANES)
IDX_PAD = VCH_PER_ROW * LANES - SEQ
ZROWS = 128
BSEG = 8
NBB = 4
BITS = (128, 64, 32, 16, 8)
DRAIN_UNITS = ROWS_PER_W * (SEQ - BSEG) // ZROWS


def _vgather16(vec, idx):
    dnums = lax.GatherDimensionNumbers(
        offset_dims=(), collapsed_slice_dims=(0,), start_index_map=(0,)
    )
    return lax.gather(
        vec, idx[:, None], dnums, slice_sizes=(1,),
        mode=lax.GatherScatterMode.PROMISE_IN_BOUNDS,
    )


def _m8(x):
    return pl.multiple_of(x, 8)


def _sc_body(len_hbm, pe_hbm, out_hbm, pos_hbm, len_v, tz_v, idx_v,
             bbufs, dummy_v, psem, dsem, bsems):
    wid = lax.axis_index("s") * NC + lax.axis_index("c")
    flat_base = _m8(wid * FLAT_PER_W)
    row_base = _m8(wid * ROWS_PER_W)

    pltpu.sync_copy(len_hbm.at[pl.ds(row_base, ROWS_PER_W)],
                    len_v.at[pl.ds(0, ROWS_PER_W)])
    pltpu.sync_copy(pe_hbm, tz_v.at[pl.ds(0, SEQ)])

    iota = lax.iota(jnp.int32, LANES)
    zero16 = jnp.zeros((LANES,), jnp.float32)

    def zrow(r, carry):
        for c in range(DCH):
            tz_v[SEQ + r, pl.ds(c * LANES, LANES)] = zero16
        return carry

    lax.fori_loop(0, ZROWS, zrow, 0)

    lo16 = len_v[pl.ds(0, LANES)]
    hi16 = len_v[pl.ds(LANES, LANES)]

    def fill_row(r, carry):
        r_lane = jnp.full((LANES,), r, jnp.int32)
        lens = jnp.where(
            r < LANES,
            _vgather16(lo16, jnp.minimum(r_lane, LANES - 1)),
            _vgather16(hi16, jnp.maximum(r_lane - LANES, 0)),
        )
        for jc in range(VCH_PER_ROW):
            j1 = jc * LANES + iota + 1
            idx_v[pl.ds(r * SEQ + jc * LANES, LANES)] = jnp.where(
                j1 <= lens, j1, 0
            )
        return carry

    lax.fori_loop(0, ROWS_PER_W, fill_row, 0)

    pltpu.async_copy(idx_v.at[pl.ds(0, FLAT_PER_W)],
                     pos_hbm.at[pl.ds(flat_base, FLAT_PER_W)], psem)

    def row_body(r, carry):
        len_s = len_v[pl.ds(r, LANES)][0]
        q8 = _m8(lax.bitwise_and(len_s, -BSEG))
        s = lax.bitwise_and(len_s, BSEG - 1)
        rowflat = _m8(flat_base + r * SEQ)

        for sz in BITS:
            off = _m8(lax.bitwise_and(q8, -(2 * sz)))

            @pl.when(lax.bitwise_and(q8, sz) != 0)
            def _():
                pltpu.async_copy(
                    tz_v.at[pl.ds(off, sz)],
                    out_hbm.at[pl.ds(_m8(rowflat + off), sz)],
                    dsem,
                )

        for bb in range(NBB):
            @pl.when(lax.rem(r, NBB) == bb)
            def _():
                @pl.when(r >= NBB)
                def _():
                    pltpu.make_async_copy(
                        bbufs[bb],
                        out_hbm.at[pl.ds(flat_base, BSEG)],
                        bsems[bb],
                    ).wait()

                for i in range(BSEG):
                    for c in range(DCH):
                        tvec = tz_v[q8 + i, pl.ds(c * LANES, LANES)]
                        bbufs[bb][i, pl.ds(c * LANES, LANES)] = jnp.where(
                            i < s, tvec, zero16
                        )
                pltpu.async_copy(
                    bbufs[bb],
                    out_hbm.at[pl.ds(_m8(rowflat + q8), BSEG)],
                    bsems[bb],
                )

        z = (SEQ - BSEG) - q8
        for sz in BITS:
            zoff = q8 + BSEG + lax.bitwise_and(z, -(2 * sz))

            @pl.when(lax.bitwise_and(z, sz) != 0)
            def _():
                pltpu.async_copy(
                    tz_v.at[pl.ds(SEQ, sz)],
                    out_hbm.at[pl.ds(_m8(rowflat + zoff), sz)],
                    dsem,
                )

        return carry

    lax.fori_loop(0, ROWS_PER_W, row_body, 0)

    def drain(i, carry):
        pltpu.make_async_copy(
            out_hbm.at[pl.ds(flat_base, ZROWS)], dummy_v, dsem
        ).wait()
        return carry

    lax.fori_loop(0, DRAIN_UNITS, drain, 0)

    for bb in range(NBB):
        pltpu.make_async_copy(
            bbufs[bb], out_hbm.at[pl.ds(flat_base, BSEG)], bsems[bb]
        ).wait()

    pltpu.make_async_copy(
        idx_v.at[pl.ds(0, FLAT_PER_W)],
        pos_hbm.at[pl.ds(flat_base, FLAT_PER_W)], psem
    ).wait()


@functools.partial(jax.jit, static_argnames=())
def _run(lens, pe_shift):
    mesh = plsc.VectorSubcoreMesh(
        core_axis_name="c", subcore_axis_name="s", num_cores=NC, num_subcores=NS
    )
    out_flat, pos_flat = pl.kernel(
        _sc_body,
        out_type=[
            jax.ShapeDtypeStruct((B * SEQ, D), jnp.float32),
            jax.ShapeDtypeStruct((B * SEQ,), jnp.int32),
        ],
        mesh=mesh,
        scratch_types=[
            pltpu.VMEM((ROWS_PER_W + LANES,), jnp.int32),
            pltpu.VMEM((SEQ + ZROWS, D), jnp.float32),
            pltpu.VMEM((FLAT_PER_W + IDX_PAD,), jnp.int32),
            [pltpu.VMEM((BSEG, D), jnp.float32) for _ in range(NBB)],
            pltpu.VMEM((ZROWS, D), jnp.float32),
            pltpu.SemaphoreType.DMA,
            pltpu.SemaphoreType.DMA,
            [pltpu.SemaphoreType.DMA for _ in range(NBB)],
        ],
    )(lens, pe_shift)
    return out_flat, pos_flat


def kernel(input_len, position_encoding):
    lens = input_len.astype(jnp.int32)
    pe_shift = position_encoding[1:]
    out_flat, pos_flat = _run(lens, pe_shift)
    positions = out_flat.reshape(B, SEQ, D)
    input_pos = pos_flat.reshape(B, SEQ)
    return positions, input_pos
